# Initial kernel scaffold; baseline (speedup 1.0000x reference)
import jax
import jax.numpy as jnp
from jax.experimental import pallas as pl


def kernel(edge_index, X, u_Y, W1, b1, W2, b2):
    raise NotImplementedError("write your pallas kernel here")


# speedup vs baseline: 12.5227x; 12.5227x over previous
import functools

import jax
import jax.numpy as jnp
from jax import lax
from jax.experimental import pallas as pl
from jax.experimental.pallas import tpu as pltpu
from jax.experimental.pallas import tpu_sc as plsc

N = 10000
E = 320000
NUM_FEATS = 96
LATENT = 32
D_IN = NUM_FEATS + LATENT
HID = 512
OUT = 2

NC = 2
NS = 16
NW = NC * NS
LN = 16
CH = 128
CPT = 79
EPT = CH * CPT
E_PAD = EPT * NW
N_PAD = 10240
ZR = N_PAD // NS
PAD_IDX = 10016
RB = 1280


def _mesh():
    return plsc.VectorSubcoreMesh(
        core_axis_name="c", subcore_axis_name="s", num_cores=NC)


@functools.lru_cache(maxsize=None)
def _make_stream_prop(width):
    @functools.partial(
        pl.kernel,
        out_type=jax.ShapeDtypeStruct((NC, N_PAD, width), jnp.float32),
        mesh=_mesh(),
        scratch_types=[
            pltpu.VMEM((CH,), jnp.int32),
            pltpu.VMEM((1, CH), jnp.int32),
            pltpu.VMEM((CH, width), jnp.float32),
            pltpu.VMEM_SHARED((N_PAD, width), jnp.float32),
            pltpu.SemaphoreType.DMA,
        ],
    )
    def prop(src_hbm, dst_hbm, table_hbm, zeros_hbm, out_hbm,
             src_v, dst_v, rows_v, acc, sem):
        cid = lax.axis_index("c")
        sid = lax.axis_index("s")
        wid = sid * NC + cid
        r0 = sid * ZR
        pltpu.sync_copy(zeros_hbm, acc.at[pl.ds(r0, ZR)])
        plsc.subcore_barrier()
        e0 = wid * EPT

        def body(i, carry):
            base = e0 + i * CH
            pltpu.sync_copy(src_hbm.at[pl.ds(base, CH)], src_v)
            pltpu.sync_copy(dst_hbm.at[pl.ds(base, CH)], dst_v.at[0])
            pltpu.async_copy(table_hbm.at[src_v], rows_v, sem).wait()
            pltpu.sync_copy(rows_v, acc.at[dst_v.at[0]], add=True)
            return carry

        lax.fori_loop(0, CPT, body, 0)
        plsc.subcore_barrier()
        pltpu.sync_copy(acc.at[pl.ds(r0, ZR)], out_hbm.at[cid, pl.ds(r0, ZR)])

    return prop


@functools.lru_cache(maxsize=None)
def _make_reg_prop(width):
    flat = N_PAD * width

    @functools.partial(
        pl.kernel,
        out_type=jax.ShapeDtypeStruct((NW, flat), jnp.float32),
        mesh=_mesh(),
        compiler_params=pltpu.CompilerParams(needs_layout_passes=False),
        scratch_types=[
            pltpu.VMEM((CH,), jnp.int32),
            pltpu.VMEM((CH,), jnp.int32),
            pltpu.VMEM((flat,), jnp.float32),
            pltpu.VMEM((flat,), jnp.float32),
        ],
    )
    def prop(src_hbm, dst_hbm, table_hbm, zeros_hbm, out_hbm,
             src_v, dst_v, table_v, acc):
        cid = lax.axis_index("c")
        sid = lax.axis_index("s")
        wid = sid * NC + cid
        pltpu.sync_copy(table_hbm, table_v)
        pltpu.sync_copy(zeros_hbm, acc)
        e0 = wid * EPT

        def body(i, carry):
            base = e0 + i * CH
            pltpu.sync_copy(src_hbm.at[pl.ds(base, CH)], src_v)
            pltpu.sync_copy(dst_hbm.at[pl.ds(base, CH)], dst_v)
            for j in range(CH // LN):
                s16 = src_v[pl.ds(j * LN, LN)] * width
                d16 = dst_v[pl.ds(j * LN, LN)] * width
                for c in range(width):
                    v = plsc.load_gather(table_v, [s16 + c])
                    plsc.addupdate_scatter(acc, [d16 + c], v)
            return carry

        lax.fori_loop(0, CPT, body, 0)
        pltpu.sync_copy(acc, out_hbm.at[wid])

    return prop


def _tc_scale(deg_parts, u_Y, X):
    def body(dp_ref, uy_ref, x_ref, s_ref, dinv_ref):
        deg = jnp.sum(dp_ref[...], axis=0) + 1.0
        dinv = lax.rsqrt(deg)
        latent = jnp.concatenate([uy_ref[...], x_ref[...]], axis=1)
        s_ref[...] = latent * dinv
        dinv_ref[...] = dinv

    return pl.pallas_call(
        body,
        grid=(N_PAD // RB,),
        in_specs=[
            pl.BlockSpec((NW, RB, 1), lambda i: (0, i, 0)),
            pl.BlockSpec((RB, LATENT), lambda i: (i, 0)),
            pl.BlockSpec((RB, NUM_FEATS), lambda i: (i, 0)),
        ],
        out_specs=[
            pl.BlockSpec((RB, D_IN), lambda i: (i, 0)),
            pl.BlockSpec((RB, 1), lambda i: (i, 0)),
        ],
        out_shape=[
            jax.ShapeDtypeStruct((N_PAD, D_IN), jnp.float32),
            jax.ShapeDtypeStruct((N_PAD, 1), jnp.float32),
        ],
    )(deg_parts, u_Y, X)


def _tc_mlp(parts, s, dinv, W1, b1r, W2):
    def body(pp_ref, s_ref, dinv_ref, w1_ref, b1_ref, w2_ref, zs_ref):
        dinv = dinv_ref[...]
        p = (pp_ref[0] + pp_ref[1] + s_ref[...]) * dinv
        h = jnp.dot(p, w1_ref[...], preferred_element_type=jnp.float32,
                    precision=lax.Precision.HIGHEST)
        h = jnp.maximum(h + b1_ref[...], 0.0)
        z = jnp.dot(h, w2_ref[...], preferred_element_type=jnp.float32,
                    precision=lax.Precision.HIGHEST)
        zs_ref[...] = z * dinv

    return pl.pallas_call(
        body,
        grid=(N_PAD // RB,),
        in_specs=[
            pl.BlockSpec((NC, RB, D_IN), lambda i: (0, i, 0)),
            pl.BlockSpec((RB, D_IN), lambda i: (i, 0)),
            pl.BlockSpec((RB, 1), lambda i: (i, 0)),
            pl.BlockSpec((D_IN, HID), lambda i: (0, 0)),
            pl.BlockSpec((1, HID), lambda i: (0, 0)),
            pl.BlockSpec((HID, OUT), lambda i: (0, 0)),
        ],
        out_specs=pl.BlockSpec((RB, OUT), lambda i: (i, 0)),
        out_shape=jax.ShapeDtypeStruct((N_PAD, OUT), jnp.float32),
    )(parts, s, dinv, W1, b1r, W2)


def _tc_softmax(qparts, zs, dinv, b2r):
    def body(q_ref, zs_ref, dinv_ref, b2_ref, out_ref):
        q = jnp.sum(q_ref[...], axis=0)
        y = (q + zs_ref[...]) * dinv_ref[...] + b2_ref[...]
        a = y[:, 0:1]
        b = y[:, 1:2]
        m = jnp.maximum(a, b)
        ea = jnp.exp(a - m)
        eb = jnp.exp(b - m)
        t = ea + eb
        out_ref[...] = jnp.concatenate([ea / t, eb / t], axis=1)

    return pl.pallas_call(
        body,
        grid=(N_PAD // RB,),
        in_specs=[
            pl.BlockSpec((NW, RB, OUT), lambda i: (0, i, 0)),
            pl.BlockSpec((RB, OUT), lambda i: (i, 0)),
            pl.BlockSpec((RB, 1), lambda i: (i, 0)),
            pl.BlockSpec((1, OUT), lambda i: (0, 0)),
        ],
        out_specs=pl.BlockSpec((RB, OUT), lambda i: (i, 0)),
        out_shape=jax.ShapeDtypeStruct((N_PAD, OUT), jnp.float32),
    )(qparts, zs, dinv, b2r)


def kernel(edge_index, X, u_Y, W1, b1, W2, b2):
    pad = jnp.full((E_PAD - E,), PAD_IDX, dtype=jnp.int32)
    srcp = jnp.concatenate([edge_index[0], pad])
    dstp = jnp.concatenate([edge_index[1], pad])

    ones1 = jnp.ones((N_PAD,), dtype=jnp.float32)
    zeros1 = jnp.zeros((N_PAD,), dtype=jnp.float32)
    zeros2 = jnp.zeros((N_PAD * OUT,), dtype=jnp.float32)
    zeros128 = jnp.zeros((ZR, D_IN), dtype=jnp.float32)

    prop1 = _make_reg_prop(1)
    prop2 = _make_reg_prop(OUT)
    prop128 = _make_stream_prop(D_IN)

    deg_parts = prop1(dstp, dstp, ones1, zeros1).reshape(NW, N_PAD, 1)
    s, dinv = _tc_scale(deg_parts, u_Y, X)
    parts = prop128(srcp, dstp, s, zeros128)
    zs = _tc_mlp(parts, s, dinv, W1, b1.reshape(1, HID), W2)
    qparts = prop2(srcp, dstp, zs.reshape(-1), zeros2).reshape(NW, N_PAD, OUT)
    y = _tc_softmax(qparts, zs, dinv, b2.reshape(1, OUT))
    return y[:N]


# speedup vs baseline: 14.5389x; 1.1610x over previous
import functools

import jax
import jax.numpy as jnp
from jax import lax
from jax.experimental import pallas as pl
from jax.experimental.pallas import tpu as pltpu
from jax.experimental.pallas import tpu_sc as plsc

N = 10000
E = 320000
NUM_FEATS = 96
LATENT = 32
D_IN = NUM_FEATS + LATENT
HID = 512
OUT = 2

NC = 2
NS = 16
NW = NC * NS
LN = 16
CH = 128
CPT = 79
EPT = CH * CPT
E_PAD = EPT * NW
N_PAD = 10240
ZR = N_PAD // NS
PAD_IDX = 10016
RB = 1280


def _mesh():
    return plsc.VectorSubcoreMesh(
        core_axis_name="c", subcore_axis_name="s", num_cores=NC)


@functools.lru_cache(maxsize=None)
def _make_stream_prop(width):
    @functools.partial(
        pl.kernel,
        out_type=jax.ShapeDtypeStruct((NC, N_PAD, width), jnp.float32),
        mesh=_mesh(),
        scratch_types=[
            pltpu.VMEM((2, CH), jnp.int32),
            pltpu.VMEM((2, CH), jnp.int32),
            pltpu.VMEM((CH, width), jnp.float32),
            pltpu.VMEM((CH, width), jnp.float32),
            pltpu.VMEM_SHARED((N_PAD, width), jnp.float32),
            pltpu.SemaphoreType.DMA,
            pltpu.SemaphoreType.DMA,
            pltpu.SemaphoreType.DMA,
            pltpu.SemaphoreType.DMA,
        ],
    )
    def prop(idx_hbm, table_hbm, zeros_hbm, out_hbm,
             ibuf0, ibuf1, rows0, rows1, acc, isem0, isem1, gsem0, gsem1):
        cid = lax.axis_index("c")
        sid = lax.axis_index("s")
        wid = sid * NC + cid
        r0 = sid * ZR
        ibuf = (ibuf0, ibuf1)
        rows = (rows0, rows1)
        isem = (isem0, isem1)
        gsem = (gsem0, gsem1)

        pltpu.async_copy(idx_hbm.at[wid, 0], ibuf0, isem0)
        pltpu.sync_copy(zeros_hbm, acc.at[pl.ds(r0, ZR)])
        plsc.subcore_barrier()
        pltpu.make_async_copy(idx_hbm.at[wid, 0], ibuf0, isem0).wait()
        pltpu.async_copy(idx_hbm.at[wid, 1], ibuf1, isem1)
        pltpu.async_copy(table_hbm.at[ibuf0.at[0]], rows0, gsem0)

        def body(k, carry):
            for b in range(2):
                i = 2 * k + b
                pltpu.make_async_copy(
                    table_hbm.at[pl.ds(0, CH)], rows[b], gsem[b]).wait()
                pltpu.make_async_copy(
                    idx_hbm.at[wid, 0], ibuf[1 - b], isem[1 - b]).wait()
                pltpu.async_copy(
                    table_hbm.at[ibuf[1 - b].at[0]], rows[1 - b], gsem[1 - b])
                pltpu.sync_copy(rows[b], acc.at[ibuf[b].at[1]], add=True)
                i2 = jnp.minimum(i + 2, CPT - 1)
                pltpu.async_copy(idx_hbm.at[wid, i2], ibuf[b], isem[b])
            return carry

        lax.fori_loop(0, (CPT - 1) // 2, body, 0)
        pltpu.make_async_copy(idx_hbm.at[wid, 0], ibuf1, isem1).wait()
        pltpu.make_async_copy(table_hbm.at[pl.ds(0, CH)], rows0, gsem0).wait()
        pltpu.sync_copy(rows0, acc.at[ibuf0.at[1]], add=True)
        plsc.subcore_barrier()
        pltpu.sync_copy(acc.at[pl.ds(r0, ZR)], out_hbm.at[cid, pl.ds(r0, ZR)])

    return prop


@functools.lru_cache(maxsize=None)
def _make_reg_prop(width):
    flat = N_PAD * width

    @functools.partial(
        pl.kernel,
        out_type=jax.ShapeDtypeStruct((NW, flat), jnp.float32),
        mesh=_mesh(),
        compiler_params=pltpu.CompilerParams(needs_layout_passes=False),
        scratch_types=[
            pltpu.VMEM((CH,), jnp.int32),
            pltpu.VMEM((CH,), jnp.int32),
            pltpu.VMEM((flat,), jnp.float32),
            pltpu.VMEM((flat,), jnp.float32),
        ],
    )
    def prop(src_hbm, dst_hbm, table_hbm, zeros_hbm, out_hbm,
             src_v, dst_v, table_v, acc):
        cid = lax.axis_index("c")
        sid = lax.axis_index("s")
        wid = sid * NC + cid
        pltpu.sync_copy(table_hbm, table_v)
        pltpu.sync_copy(zeros_hbm, acc)
        e0 = wid * EPT

        def body(i, carry):
            base = e0 + i * CH
            pltpu.sync_copy(src_hbm.at[pl.ds(base, CH)], src_v)
            pltpu.sync_copy(dst_hbm.at[pl.ds(base, CH)], dst_v)
            for j in range(CH // LN):
                s16 = src_v[pl.ds(j * LN, LN)] * width
                d16 = dst_v[pl.ds(j * LN, LN)] * width
                for c in range(width):
                    v = plsc.load_gather(table_v, [s16 + c])
                    plsc.addupdate_scatter(acc, [d16 + c], v)
            return carry

        lax.fori_loop(0, CPT, body, 0)
        pltpu.sync_copy(acc, out_hbm.at[wid])

    return prop


def _tc_scale(deg_parts, u_Y, X):
    def body(dp_ref, uy_ref, x_ref, s_ref, dinv_ref):
        deg = jnp.sum(dp_ref[...], axis=0) + 1.0
        dinv = lax.rsqrt(deg)
        latent = jnp.concatenate([uy_ref[...], x_ref[...]], axis=1)
        s_ref[...] = latent * dinv
        dinv_ref[...] = dinv

    return pl.pallas_call(
        body,
        grid=(N_PAD // RB,),
        in_specs=[
            pl.BlockSpec((NW, RB, 1), lambda i: (0, i, 0)),
            pl.BlockSpec((RB, LATENT), lambda i: (i, 0)),
            pl.BlockSpec((RB, NUM_FEATS), lambda i: (i, 0)),
        ],
        out_specs=[
            pl.BlockSpec((RB, D_IN), lambda i: (i, 0)),
            pl.BlockSpec((RB, 1), lambda i: (i, 0)),
        ],
        out_shape=[
            jax.ShapeDtypeStruct((N_PAD, D_IN), jnp.float32),
            jax.ShapeDtypeStruct((N_PAD, 1), jnp.float32),
        ],
    )(deg_parts, u_Y, X)


def _tc_mlp(parts, s, dinv, W1, b1r, W2):
    def body(pp_ref, s_ref, dinv_ref, w1_ref, b1_ref, w2_ref, zs_ref):
        dinv = dinv_ref[...]
        p = (pp_ref[0] + pp_ref[1] + s_ref[...]) * dinv
        h = jnp.dot(p, w1_ref[...], preferred_element_type=jnp.float32,
                    precision=lax.Precision.HIGHEST)
        h = jnp.maximum(h + b1_ref[...], 0.0)
        z = jnp.dot(h, w2_ref[...], preferred_element_type=jnp.float32,
                    precision=lax.Precision.HIGHEST)
        zs_ref[...] = z * dinv

    return pl.pallas_call(
        body,
        grid=(N_PAD // RB,),
        in_specs=[
            pl.BlockSpec((NC, RB, D_IN), lambda i: (0, i, 0)),
            pl.BlockSpec((RB, D_IN), lambda i: (i, 0)),
            pl.BlockSpec((RB, 1), lambda i: (i, 0)),
            pl.BlockSpec((D_IN, HID), lambda i: (0, 0)),
            pl.BlockSpec((1, HID), lambda i: (0, 0)),
            pl.BlockSpec((HID, OUT), lambda i: (0, 0)),
        ],
        out_specs=pl.BlockSpec((RB, OUT), lambda i: (i, 0)),
        out_shape=jax.ShapeDtypeStruct((N_PAD, OUT), jnp.float32),
    )(parts, s, dinv, W1, b1r, W2)


def _tc_softmax(qparts, zs, dinv, b2r):
    def body(q_ref, zs_ref, dinv_ref, b2_ref, out_ref):
        q = jnp.sum(q_ref[...], axis=0)
        y = (q + zs_ref[...]) * dinv_ref[...] + b2_ref[...]
        a = y[:, 0:1]
        b = y[:, 1:2]
        m = jnp.maximum(a, b)
        ea = jnp.exp(a - m)
        eb = jnp.exp(b - m)
        t = ea + eb
        out_ref[...] = jnp.concatenate([ea / t, eb / t], axis=1)

    return pl.pallas_call(
        body,
        grid=(N_PAD // RB,),
        in_specs=[
            pl.BlockSpec((NW, RB, OUT), lambda i: (0, i, 0)),
            pl.BlockSpec((RB, OUT), lambda i: (i, 0)),
            pl.BlockSpec((RB, 1), lambda i: (i, 0)),
            pl.BlockSpec((1, OUT), lambda i: (0, 0)),
        ],
        out_specs=pl.BlockSpec((RB, OUT), lambda i: (i, 0)),
        out_shape=jax.ShapeDtypeStruct((N_PAD, OUT), jnp.float32),
    )(qparts, zs, dinv, b2r)


def kernel(edge_index, X, u_Y, W1, b1, W2, b2):
    pad = jnp.full((E_PAD - E,), PAD_IDX, dtype=jnp.int32)
    srcp = jnp.concatenate([edge_index[0], pad])
    dstp = jnp.concatenate([edge_index[1], pad])

    ones1 = jnp.ones((N_PAD,), dtype=jnp.float32)
    zeros1 = jnp.zeros((N_PAD,), dtype=jnp.float32)
    zeros2 = jnp.zeros((N_PAD * OUT,), dtype=jnp.float32)
    zeros128 = jnp.zeros((ZR, D_IN), dtype=jnp.float32)

    prop1 = _make_reg_prop(1)
    prop2 = _make_reg_prop(OUT)
    prop128 = _make_stream_prop(D_IN)

    deg_parts = prop1(dstp, dstp, ones1, zeros1).reshape(NW, N_PAD, 1)
    s, dinv = _tc_scale(deg_parts, u_Y, X)
    idx_packed = jnp.stack(
        [srcp.reshape(NW, CPT, CH), dstp.reshape(NW, CPT, CH)], axis=2)
    parts = prop128(idx_packed, s, zeros128)
    zs = _tc_mlp(parts, s, dinv, W1, b1.reshape(1, HID), W2)
    qparts = prop2(srcp, dstp, zs.reshape(-1), zeros2).reshape(NW, N_PAD, OUT)
    y = _tc_softmax(qparts, zs, dinv, b2.reshape(1, OUT))
    return y[:N]
